# Initial kernel scaffold; baseline (speedup 1.0000x reference)
import jax
import jax.numpy as jnp
from jax.experimental import pallas as pl


def kernel(pred_logits, pred_boxes, tgt_labels, tgt_boxes, image_size_xyxy, image_size_xyxy_tgt):
    raise NotImplementedError("write your pallas kernel here")


# speedup vs baseline: 3.4599x; 3.4599x over previous
import functools

import jax
import jax.numpy as jnp
from jax.experimental import pallas as pl
from jax.experimental.pallas import tpu as pltpu

_COST_CLASS = 2.0
_COST_BBOX = 5.0
_COST_GIOU = 2.0

_BLOCK_R = 320
_BLOCK_C = 2048
_LANES = 128


def _cost_kernel(logits_ref, prow_ref, tcol_ref, out_ref, *, block_c):
    x = logits_ref[...]
    m = jnp.max(x, axis=1, keepdims=True)
    e = jnp.exp(x - m)
    s = jnp.sum(e, axis=1, keepdims=True)
    q = e * (-_COST_CLASS / s)
    lane = jax.lax.broadcasted_iota(jnp.int32, (1, _LANES), 1)
    q = jnp.where(lane == _LANES - 1, 2.0, q)

    ids = tcol_ref[8:9, :].astype(jnp.int32)
    cls = jax.lax.broadcasted_iota(jnp.int32, (_LANES, block_c), 0)
    sel = jnp.logical_or(cls == ids, cls == _LANES - 1)
    sel = sel.astype(jnp.float32)
    acc = jnp.dot(q, sel, preferred_element_type=jnp.float32)

    pr = prow_ref[...]
    tc = tcol_ref[...]

    for c in range(4):
        a = (_COST_BBOX * pr[:, c:c + 1]) * pr[:, c + 4:c + 5]
        b = (_COST_BBOX * tc[c:c + 1, :]) * tc[c + 4:c + 5, :]
        acc = acc + jnp.abs(a - b)

    ax1 = pr[:, 0:1]
    ay1 = pr[:, 1:2]
    ax2 = pr[:, 2:3]
    ay2 = pr[:, 3:4]
    bx1 = tc[0:1, :]
    by1 = tc[1:2, :]
    bx2 = tc[2:3, :]
    by2 = tc[3:4, :]
    area_a = (ax2 - ax1) * (ay2 - ay1)
    area_b = (bx2 - bx1) * (by2 - by1)

    max_x1 = jnp.maximum(ax1, bx1)
    min_x1 = jnp.minimum(ax1, bx1)
    max_x2 = jnp.maximum(ax2, bx2)
    min_x2 = jnp.minimum(ax2, bx2)
    max_y1 = jnp.maximum(ay1, by1)
    min_y1 = jnp.minimum(ay1, by1)
    max_y2 = jnp.maximum(ay2, by2)
    min_y2 = jnp.minimum(ay2, by2)

    iw = jnp.maximum(min_x2 - max_x1, 0.0)
    ih = jnp.maximum(min_y2 - max_y1, 0.0)
    inter = iw * ih
    union = (area_a + area_b) - inter
    area_e = (max_x2 - min_x1) * (max_y2 - min_y1)

    acc = acc + (-_COST_GIOU * inter) / union
    acc = acc + (-_COST_GIOU * union) / area_e
    out_ref[...] = acc


def kernel(pred_logits, pred_boxes, tgt_labels, tgt_boxes,
           image_size_xyxy, image_size_xyxy_tgt):
    b, q, nc = pred_logits.shape
    t = tgt_labels.shape[1]
    bq = b * q
    bt = b * t

    f32 = jnp.float32
    logits = pred_logits.reshape(bq, nc).astype(f32)
    logits_p = jnp.pad(logits, ((0, 0), (0, _LANES - nc)),
                       constant_values=-1e30)

    inv_img = 1.0 / image_size_xyxy
    prow = jnp.concatenate(
        [pred_boxes, jnp.broadcast_to(inv_img[:, None, :], (b, q, 4))],
        axis=-1).reshape(bq, 8).astype(f32)

    inv_tgt = 1.0 / image_size_xyxy_tgt
    tcol8 = jnp.concatenate([tgt_boxes, inv_tgt], axis=-1)
    tcol8 = tcol8.reshape(bt, 8).T
    ids_row = tgt_labels.reshape(1, bt).astype(f32)
    tcol = jnp.concatenate(
        [tcol8, ids_row, jnp.zeros((7, bt), f32)], axis=0)

    grid = (bq // _BLOCK_R, bt // _BLOCK_C)
    out = pl.pallas_call(
        functools.partial(_cost_kernel, block_c=_BLOCK_C),
        out_shape=jax.ShapeDtypeStruct((bq, bt), f32),
        grid=grid,
        in_specs=[
            pl.BlockSpec((_BLOCK_R, _LANES), lambda i, j: (i, 0)),
            pl.BlockSpec((_BLOCK_R, 8), lambda i, j: (i, 0)),
            pl.BlockSpec((16, _BLOCK_C), lambda i, j: (0, j)),
        ],
        out_specs=pl.BlockSpec((_BLOCK_R, _BLOCK_C), lambda i, j: (i, j)),
        compiler_params=pltpu.CompilerParams(
            dimension_semantics=("parallel", "arbitrary"),
            vmem_limit_bytes=100 * 1024 * 1024,
        ),
        name="hungarian_cost",
    )(logits_p, prow, tcol)

    return out.reshape(b, q, bt)


# speedup vs baseline: 4.5870x; 1.3257x over previous
import functools

import jax
import jax.numpy as jnp
from jax.experimental import pallas as pl
from jax.experimental.pallas import tpu as pltpu

_COST_CLASS = 2.0
_COST_BBOX = 5.0
_COST_GIOU = 2.0

_BLOCK_C = 2048
_LANES = 128


def _cost_kernel(logits_ref, prow_ref, tcol_ref, out_ref, *, block_c):
    x = logits_ref[0]
    m = jnp.max(x, axis=1, keepdims=True)
    e = jnp.exp(x - m)
    s = jnp.sum(e, axis=1, keepdims=True)
    q = e * (-_COST_CLASS / s)
    lane = jax.lax.broadcasted_iota(jnp.int32, (1, _LANES), 1)
    q = jnp.where(lane == _LANES - 1, 2.0, q)

    ids = tcol_ref[8:9, :].astype(jnp.int32)
    cls = jax.lax.broadcasted_iota(jnp.int32, (_LANES, block_c), 0)
    sel = jnp.logical_or(cls == ids, cls == _LANES - 1)
    sel = sel.astype(jnp.float32)
    acc = jnp.dot(q, sel, preferred_element_type=jnp.float32)

    pr = prow_ref[0]
    tc = tcol_ref[...]

    for c in range(4):
        a = (_COST_BBOX * pr[:, c:c + 1]) * pr[:, c + 4:c + 5]
        b = (_COST_BBOX * tc[c:c + 1, :]) * tc[c + 4:c + 5, :]
        acc = acc + jnp.abs(a - b)

    ax1 = pr[:, 0:1]
    ay1 = pr[:, 1:2]
    ax2 = pr[:, 2:3]
    ay2 = pr[:, 3:4]
    bx1 = tc[0:1, :]
    by1 = tc[1:2, :]
    bx2 = tc[2:3, :]
    by2 = tc[3:4, :]
    area_a = (ax2 - ax1) * (ay2 - ay1)
    area_b = (bx2 - bx1) * (by2 - by1)

    max_x1 = jnp.maximum(ax1, bx1)
    min_x1 = jnp.minimum(ax1, bx1)
    max_x2 = jnp.maximum(ax2, bx2)
    min_x2 = jnp.minimum(ax2, bx2)
    max_y1 = jnp.maximum(ay1, by1)
    min_y1 = jnp.minimum(ay1, by1)
    max_y2 = jnp.maximum(ay2, by2)
    min_y2 = jnp.minimum(ay2, by2)

    iw = jnp.maximum(min_x2 - max_x1, 0.0)
    ih = jnp.maximum(min_y2 - max_y1, 0.0)
    inter = iw * ih
    union = (area_a + area_b) - inter
    area_e = (max_x2 - min_x1) * (max_y2 - min_y1)

    acc = acc + (-_COST_GIOU * inter) / union
    acc = acc + (-_COST_GIOU * union) / area_e
    out_ref[0] = acc


def kernel(pred_logits, pred_boxes, tgt_labels, tgt_boxes,
           image_size_xyxy, image_size_xyxy_tgt):
    b, q, nc = pred_logits.shape
    t = tgt_labels.shape[1]
    bt = b * t

    f32 = jnp.float32
    logits_p = jnp.pad(pred_logits.astype(f32),
                       ((0, 0), (0, 0), (0, _LANES - nc)),
                       constant_values=-1e30)

    inv_img = 1.0 / image_size_xyxy
    prow = jnp.concatenate(
        [pred_boxes, jnp.broadcast_to(inv_img[:, None, :], (b, q, 4))],
        axis=-1).astype(f32)

    inv_tgt = 1.0 / image_size_xyxy_tgt
    tcol8 = jnp.concatenate([tgt_boxes, inv_tgt], axis=-1)
    tcol8 = tcol8.reshape(bt, 8).T
    ids_row = tgt_labels.reshape(1, bt).astype(f32)
    tcol = jnp.concatenate(
        [tcol8, ids_row, jnp.zeros((7, bt), f32)], axis=0)

    grid = (b, bt // _BLOCK_C)
    out = pl.pallas_call(
        functools.partial(_cost_kernel, block_c=_BLOCK_C),
        out_shape=jax.ShapeDtypeStruct((b, q, bt), f32),
        grid=grid,
        in_specs=[
            pl.BlockSpec((1, q, _LANES), lambda i, j: (i, 0, 0)),
            pl.BlockSpec((1, q, 8), lambda i, j: (i, 0, 0)),
            pl.BlockSpec((16, _BLOCK_C), lambda i, j: (0, j)),
        ],
        out_specs=pl.BlockSpec((1, q, _BLOCK_C), lambda i, j: (i, 0, j)),
        compiler_params=pltpu.CompilerParams(
            dimension_semantics=("parallel", "arbitrary"),
            vmem_limit_bytes=60 * 1024 * 1024,
        ),
        name="hungarian_cost",
    )(logits_p, prow, tcol)

    return out
